# baseline (device time: 7082 ns/iter reference)
import jax
import jax.numpy as jnp
from jax import lax
from jax.experimental import pallas as pl
from jax.experimental.pallas import tpu as pltpu


def kernel(x, dy, gamma):
    m, d = x.shape

    def body(x_ref, dy_ref, out_ref, send_buf, recv_buf, send_sem, recv_sem):
        my_x = lax.axis_index("x")
        my_y = lax.axis_index("y")
        peer = (1 - my_x, my_y)

        barrier_sem = pltpu.get_barrier_semaphore()
        pl.semaphore_signal(
            barrier_sem, inc=2, device_id=peer,
            device_id_type=pl.DeviceIdType.MESH,
        )

        xf = x_ref[:, :]
        dyf = dy_ref[:, :]
        mu = jnp.mean(xf, axis=1, keepdims=True)
        ex2 = jnp.mean(xf * xf, axis=1, keepdims=True)
        rstd = lax.rsqrt(ex2 - mu * mu + 1e-5)
        xhat = (xf - mu) * rstd
        send_buf[0, :] = jnp.sum(dyf * xhat, axis=0)
        send_buf[1, :] = jnp.sum(dyf, axis=0)

        pl.semaphore_wait(barrier_sem, 2)

        rdma = pltpu.make_async_remote_copy(
            src_ref=send_buf, dst_ref=recv_buf,
            send_sem=send_sem, recv_sem=recv_sem,
            device_id=peer, device_id_type=pl.DeviceIdType.MESH,
        )
        rdma.start()
        rdma.wait()

        out_ref[:, :] = send_buf[:, :] + recv_buf[:, :]

    return pl.pallas_call(
        body,
        out_shape=jax.ShapeDtypeStruct((2, d), jnp.float32),
        in_specs=[
            pl.BlockSpec(memory_space=pltpu.MemorySpace.VMEM),
            pl.BlockSpec(memory_space=pltpu.MemorySpace.VMEM),
        ],
        out_specs=pl.BlockSpec(memory_space=pltpu.MemorySpace.VMEM),
        scratch_shapes=[
            pltpu.VMEM((2, d), jnp.float32),
            pltpu.VMEM((2, d), jnp.float32),
            pltpu.SemaphoreType.DMA,
            pltpu.SemaphoreType.DMA,
        ],
        compiler_params=pltpu.CompilerParams(collective_id=9),
    )(x, dy)
